# baseline (speedup 1.0000x reference)
import jax
import jax.numpy as jnp
from jax import lax
from jax.experimental import pallas as pl
from jax.experimental.pallas import tpu as pltpu
from jax.experimental.pallas import tpu_sc as plsc

N = 10000
E = 320000
D = 128
DH1 = 180
NCLS = 40

NC, NS = 2, 16
NW = NC * NS
EPW = E // NW
CH = 128
NCH = 80
PHASES = 2
PCH = NCH // PHASES
EPP = PCH * CH
EPAD = NCH * CH
NROWS = N + 112
RPT = NROWS // NS

_mesh = plsc.VectorSubcoreMesh(
    core_axis_name="c", subcore_axis_name="s", num_cores=NC, num_subcores=NS
)


HR = EPAD // 128


def _deg_body(dst_hbm, zeros_hbm, out_hbm, dst_v, hist_v, idx_v, accum):
    c = lax.axis_index("c")
    s = lax.axis_index("s")
    wid = c * NS + s
    ones = jnp.full((16,), 1.0, jnp.float32)
    pltpu.sync_copy(zeros_hbm.at[pl.ds(0, HR)], hist_v)
    for k in range(HR // 16):
        idx_v[pl.ds(16 * k, 16)] = lax.iota(jnp.int32, 16) + 16 * k

    @pl.when(s == 0)
    def _():
        pltpu.sync_copy(zeros_hbm.at[pl.ds(0, HR)], accum)

    plsc.subcore_barrier()

    for ph in range(PHASES):
        pltpu.sync_copy(dst_hbm.at[wid, ph], dst_v)

        def step(i, carry):
            for u in range(4):
                iv = dst_v[pl.ds((i * 4 + u) * 16, 16)]
                row = jax.lax.shift_right_logical(iv, 7)
                col = jnp.bitwise_and(iv, 127)
                plsc.addupdate_scatter(hist_v, [row, col], ones)
            return carry

        lax.fori_loop(0, EPP // 64, step, 0)

    pltpu.sync_copy(hist_v, accum.at[idx_v], add=True)
    plsc.subcore_barrier()

    @pl.when(s == 0)
    def _():
        pltpu.sync_copy(accum, out_hbm.at[c])


_deg_pass = pl.kernel(
    _deg_body,
    out_type=jax.ShapeDtypeStruct((NC, HR, 128), jnp.float32),
    mesh=_mesh,
    compiler_params=pltpu.CompilerParams(needs_layout_passes=False,
                                         skip_device_barrier=True),
    scratch_types=[
        pltpu.VMEM((EPP,), jnp.int32),
        pltpu.VMEM((HR, 128), jnp.float32),
        pltpu.VMEM((HR,), jnp.int32),
        pltpu.VMEM_SHARED((HR, 128), jnp.float32),
    ],
)


def _edge_body(h_hbm, src_hbm, dst_hbm, zeros_hbm, out_hbm,
               src_v, dst_v, buf0, buf1, accum, sem0, sem1, zsem):
    c = lax.axis_index("c")
    s = lax.axis_index("s")
    wid = c * NS + s
    r0 = s * RPT
    zcp = pltpu.async_copy(zeros_hbm.at[pl.ds(r0, RPT)],
                           accum.at[pl.ds(r0, RPT)], zsem)
    pltpu.sync_copy(src_hbm.at[wid, 0], src_v)
    pltpu.sync_copy(dst_hbm.at[wid, 0], dst_v)
    zcp.wait()
    plsc.subcore_barrier()

    for ph in range(PHASES):
        if ph > 0:
            pltpu.sync_copy(src_hbm.at[wid, ph], src_v)
            pltpu.sync_copy(dst_hbm.at[wid, ph], dst_v)

        pltpu.async_copy(h_hbm.at[src_v.at[pl.ds(0, CH)]], buf0, sem0)
        pltpu.async_copy(h_hbm.at[src_v.at[pl.ds(CH, CH)]], buf1, sem1)

        def step(j, carry):
            jj = 2 * j
            pltpu.make_async_copy(
                h_hbm.at[src_v.at[pl.ds(jj * CH, CH)]], buf0, sem0).wait()
            pltpu.sync_copy(buf0, accum.at[dst_v.at[jj]], add=True)
            pltpu.async_copy(
                h_hbm.at[src_v.at[pl.ds((jj + 2) * CH, CH)]], buf0, sem0)
            pltpu.make_async_copy(
                h_hbm.at[src_v.at[pl.ds((jj + 1) * CH, CH)]], buf1, sem1).wait()
            pltpu.sync_copy(buf1, accum.at[dst_v.at[jj + 1]], add=True)
            pltpu.async_copy(
                h_hbm.at[src_v.at[pl.ds((jj + 3) * CH, CH)]], buf1, sem1)
            return carry

        lax.fori_loop(0, PCH // 2 - 1, step, 0)

        jl = PCH - 2
        pltpu.make_async_copy(
            h_hbm.at[src_v.at[pl.ds(jl * CH, CH)]], buf0, sem0).wait()
        pltpu.sync_copy(buf0, accum.at[dst_v.at[jl]], add=True)
        pltpu.make_async_copy(
            h_hbm.at[src_v.at[pl.ds((jl + 1) * CH, CH)]], buf1, sem1).wait()
        pltpu.sync_copy(buf1, accum.at[dst_v.at[jl + 1]], add=True)

    plsc.subcore_barrier()
    pltpu.sync_copy(accum.at[pl.ds(r0, RPT)], out_hbm.at[c, pl.ds(r0, RPT)])


_edge_pass = pl.kernel(
    _edge_body,
    out_type=jax.ShapeDtypeStruct((NC, NROWS, D), jnp.float32),
    mesh=_mesh,
    compiler_params=pltpu.CompilerParams(skip_device_barrier=True),
    scratch_types=[
        pltpu.VMEM((EPP,), jnp.int32),
        pltpu.VMEM((PCH, CH), jnp.int32),
        pltpu.VMEM((CH, D), jnp.float32),
        pltpu.VMEM((CH, D), jnp.float32),
        pltpu.VMEM_SHARED((NROWS, D), jnp.float32),
        pltpu.SemaphoreType.DMA,
        pltpu.SemaphoreType.DMA,
        pltpu.SemaphoreType.DMA,
    ],
)


R = 10000
G = N // R


def _dinv_from(degp_ref):
    deg = degp_ref[0, :, 0] + degp_ref[1, :, 0] + 1.0
    return lax.rsqrt(deg)


def _tc1_body(x_ref, degp_ref, y_ref):
    dinv = _dinv_from(degp_ref)
    y_ref[...] = x_ref[...] * dinv[:, None]


def _tc2_body(p_ref, y_ref, degp_ref, w1_ref, b1_ref, w2_ref, out_ref):
    dinv = _dinv_from(degp_ref)
    s1 = (p_ref[0] + p_ref[1] + y_ref[...]) * dinv[:, None]
    z1 = jnp.dot(s1, w1_ref[...], preferred_element_type=jnp.float32) + b1_ref[...]
    a1 = jnp.where(z1 >= 0, z1, 0.01 * z1)
    out_ref[...] = (
        jnp.dot(a1, w2_ref[...], preferred_element_type=jnp.float32) * dinv[:, None]
    )


def _tc3_body(p_ref, h2_ref, degp_ref, b2_ref, wl_ref, bl_ref, out_ref):
    dinv = _dinv_from(degp_ref)
    z = (p_ref[0] + p_ref[1] + h2_ref[...]) * dinv[:, None] + b2_ref[...]
    a = jnp.where(z >= 0, z, 0.01 * z)
    logits = jnp.dot(a, wl_ref[...], preferred_element_type=jnp.float32) + bl_ref[...]
    m = jnp.max(logits, axis=1, keepdims=True)
    lse = jnp.log(jnp.sum(jnp.exp(logits - m), axis=1, keepdims=True)) + m
    out_ref[...] = logits - lse


def _tc1(x, degp):
    return pl.pallas_call(
        _tc1_body,
        grid=(G,),
        in_specs=[
            pl.BlockSpec((R, 128), lambda i: (i, 0)),
            pl.BlockSpec((NC, R, 1), lambda i: (0, i, 0)),
        ],
        out_specs=pl.BlockSpec((R, D), lambda i: (i, 0)),
        out_shape=jax.ShapeDtypeStruct((N, D), jnp.float32),
    )(x, degp)


def _tc2(p1, y, degp, w1, b1r, w2p):
    return pl.pallas_call(
        _tc2_body,
        grid=(G,),
        in_specs=[
            pl.BlockSpec((NC, R, D), lambda i: (0, i, 0)),
            pl.BlockSpec((R, D), lambda i: (i, 0)),
            pl.BlockSpec((NC, R, 1), lambda i: (0, i, 0)),
            pl.BlockSpec((128, DH1), lambda i: (0, 0)),
            pl.BlockSpec((1, DH1), lambda i: (0, 0)),
            pl.BlockSpec((DH1, D), lambda i: (0, 0)),
        ],
        out_specs=pl.BlockSpec((R, D), lambda i: (i, 0)),
        out_shape=jax.ShapeDtypeStruct((N, D), jnp.float32),
    )(p1, y, degp, w1, b1r, w2p)


def _tc3(p2, h2, degp, b2p, wlp, blp):
    return pl.pallas_call(
        _tc3_body,
        grid=(G,),
        in_specs=[
            pl.BlockSpec((NC, R, D), lambda i: (0, i, 0)),
            pl.BlockSpec((R, D), lambda i: (i, 0)),
            pl.BlockSpec((NC, R, 1), lambda i: (0, i, 0)),
            pl.BlockSpec((1, D), lambda i: (0, 0)),
            pl.BlockSpec((D, NCLS), lambda i: (0, 0)),
            pl.BlockSpec((1, NCLS), lambda i: (0, 0)),
        ],
        out_specs=pl.BlockSpec((R, NCLS), lambda i: (i, 0)),
        out_shape=jax.ShapeDtypeStruct((N, NCLS), jnp.float32),
    )(p2, h2, degp, b2p, wlp, blp)


def kernel(x, edge_index, W1, b1, W2, b2, Wl, bl):
    src = edge_index[0].reshape(NW, EPW)
    dst = edge_index[1].reshape(NW, EPW)
    npad = EPAD - EPW
    padc = jnp.arange(npad, dtype=jnp.int32) % 16
    src_g = jnp.concatenate(
        [src, jnp.broadcast_to(padc, (NW, npad))], axis=1
    ).reshape(NW, PHASES, EPP)
    dst_g = jnp.concatenate(
        [dst, jnp.broadcast_to(N + padc, (NW, npad))], axis=1
    ).reshape(NW, PHASES, PCH, CH)

    dst_f = dst_g.reshape(NW, PHASES, EPP)
    zacc = jnp.zeros((NROWS, D), jnp.float32)

    b1r = b1.reshape(1, DH1)
    w2p = jnp.pad(W2, ((0, 0), (0, D - W2.shape[1])))
    b2p = jnp.pad(b2, (0, D - b2.shape[0])).reshape(1, D)
    wlp = jnp.pad(Wl, ((0, D - Wl.shape[0]), (0, 0)))
    blp = bl.reshape(1, NCLS)

    degh = _deg_pass(dst_f, zacc)
    degp = degh.reshape(NC, HR * 128)[:, :N].reshape(NC, N, 1)
    y = _tc1(x, degp)
    p1 = _edge_pass(y, src_g, dst_g, zacc)
    h2 = _tc2(p1, y, degp, W1, b1r, w2p)
    p2 = _edge_pass(h2, src_g, dst_g, zacc)
    return _tc3(p2, h2, degp, b2p, wlp, blp)


# speedup vs baseline: 1.0150x; 1.0150x over previous
import jax
import jax.numpy as jnp
from jax import lax
from jax.experimental import pallas as pl
from jax.experimental.pallas import tpu as pltpu
from jax.experimental.pallas import tpu_sc as plsc

N = 10000
E = 320000
D = 128
DH1 = 180
NCLS = 40

NC, NS = 2, 16
NW = NC * NS
EPW = E // NW
CH = 128
NCH = 80
PHASES = 2
PCH = NCH // PHASES
EPP = PCH * CH
EPAD = NCH * CH
NROWS = N + 112
RPT = NROWS // NS

_mesh = plsc.VectorSubcoreMesh(
    core_axis_name="c", subcore_axis_name="s", num_cores=NC, num_subcores=NS
)


HR = EPAD // 128


def _deg_body(dst_hbm, zeros_hbm, out_hbm, dst_v, hist_v, idx_v, accum):
    c = lax.axis_index("c")
    s = lax.axis_index("s")
    wid = c * NS + s
    ones = jnp.full((16,), 1.0, jnp.float32)
    pltpu.sync_copy(zeros_hbm.at[pl.ds(0, HR)], hist_v)
    for k in range(HR // 16):
        idx_v[pl.ds(16 * k, 16)] = lax.iota(jnp.int32, 16) + 16 * k

    @pl.when(s == 0)
    def _():
        pltpu.sync_copy(zeros_hbm.at[pl.ds(0, HR)], accum)

    plsc.subcore_barrier()

    for ph in range(PHASES):
        pltpu.sync_copy(dst_hbm.at[wid, ph], dst_v)

        def step(i, carry):
            for u in range(4):
                iv = dst_v[pl.ds((i * 4 + u) * 16, 16)]
                row = jax.lax.shift_right_logical(iv, 7)
                col = jnp.bitwise_and(iv, 127)
                plsc.addupdate_scatter(hist_v, [row, col], ones)
            return carry

        lax.fori_loop(0, EPP // 64, step, 0)

    pltpu.sync_copy(hist_v, accum.at[idx_v], add=True)
    plsc.subcore_barrier()

    @pl.when(s == 0)
    def _():
        pltpu.sync_copy(accum, out_hbm.at[c])


_deg_pass = pl.kernel(
    _deg_body,
    out_type=jax.ShapeDtypeStruct((NC, HR, 128), jnp.float32),
    mesh=_mesh,
    compiler_params=pltpu.CompilerParams(needs_layout_passes=False,
                                         skip_device_barrier=True),
    scratch_types=[
        pltpu.VMEM((EPP,), jnp.int32),
        pltpu.VMEM((HR, 128), jnp.float32),
        pltpu.VMEM((HR,), jnp.int32),
        pltpu.VMEM_SHARED((HR, 128), jnp.float32),
    ],
)


def _edge_body(h_hbm, src_hbm, dst_hbm, zeros_hbm, out_hbm,
               src_v, dst_v, buf0, buf1, accum, sem0, sem1, zsem):
    c = lax.axis_index("c")
    s = lax.axis_index("s")
    wid = c * NS + s
    r0 = s * RPT
    zcp = pltpu.async_copy(zeros_hbm.at[pl.ds(r0, RPT)],
                           accum.at[pl.ds(r0, RPT)], zsem)
    pltpu.sync_copy(src_hbm.at[wid, 0], src_v)
    pltpu.sync_copy(dst_hbm.at[wid, 0], dst_v)
    zcp.wait()
    plsc.subcore_barrier()

    for ph in range(PHASES):
        if ph > 0:
            pltpu.sync_copy(src_hbm.at[wid, ph], src_v)
            pltpu.sync_copy(dst_hbm.at[wid, ph], dst_v)

        pltpu.async_copy(h_hbm.at[src_v.at[pl.ds(0, CH)]], buf0, sem0)
        pltpu.async_copy(h_hbm.at[src_v.at[pl.ds(CH, CH)]], buf1, sem1)

        def step(j, carry):
            jj = 2 * j
            pltpu.make_async_copy(
                h_hbm.at[src_v.at[pl.ds(jj * CH, CH)]], buf0, sem0).wait()
            pltpu.sync_copy(buf0, accum.at[dst_v.at[jj]], add=True)
            pltpu.async_copy(
                h_hbm.at[src_v.at[pl.ds((jj + 2) * CH, CH)]], buf0, sem0)
            pltpu.make_async_copy(
                h_hbm.at[src_v.at[pl.ds((jj + 1) * CH, CH)]], buf1, sem1).wait()
            pltpu.sync_copy(buf1, accum.at[dst_v.at[jj + 1]], add=True)
            pltpu.async_copy(
                h_hbm.at[src_v.at[pl.ds((jj + 3) * CH, CH)]], buf1, sem1)
            return carry

        lax.fori_loop(0, PCH // 2 - 1, step, 0)

        jl = PCH - 2
        pltpu.make_async_copy(
            h_hbm.at[src_v.at[pl.ds(jl * CH, CH)]], buf0, sem0).wait()
        pltpu.sync_copy(buf0, accum.at[dst_v.at[jl]], add=True)
        pltpu.make_async_copy(
            h_hbm.at[src_v.at[pl.ds((jl + 1) * CH, CH)]], buf1, sem1).wait()
        pltpu.sync_copy(buf1, accum.at[dst_v.at[jl + 1]], add=True)

    plsc.subcore_barrier()
    pltpu.sync_copy(accum.at[pl.ds(r0, RPT)], out_hbm.at[c, pl.ds(r0, RPT)])


_edge_pass = pl.kernel(
    _edge_body,
    out_type=jax.ShapeDtypeStruct((NC, NROWS, D), jnp.float32),
    mesh=_mesh,
    compiler_params=pltpu.CompilerParams(skip_device_barrier=True),
    scratch_types=[
        pltpu.VMEM((EPP,), jnp.int32),
        pltpu.VMEM((PCH, CH), jnp.int32),
        pltpu.VMEM((CH, D), jnp.float32),
        pltpu.VMEM((CH, D), jnp.float32),
        pltpu.VMEM_SHARED((NROWS, D), jnp.float32),
        pltpu.SemaphoreType.DMA,
        pltpu.SemaphoreType.DMA,
        pltpu.SemaphoreType.DMA,
    ],
)


R = 5000
G = N // R


def _dinv_from(degp_ref):
    deg = degp_ref[0, :, 0] + degp_ref[1, :, 0] + 1.0
    return lax.rsqrt(deg)


def _tc1_body(x_ref, degp_ref, y_ref):
    dinv = _dinv_from(degp_ref)
    y_ref[...] = x_ref[...] * dinv[:, None]


def _tc2_body(p_ref, y_ref, degp_ref, w1_ref, b1_ref, w2_ref, out_ref):
    dinv = _dinv_from(degp_ref)
    s1 = (p_ref[0] + p_ref[1] + y_ref[...]) * dinv[:, None]
    z1 = jnp.dot(s1, w1_ref[...], preferred_element_type=jnp.float32) + b1_ref[...]
    a1 = jnp.where(z1 >= 0, z1, 0.01 * z1)
    out_ref[...] = (
        jnp.dot(a1, w2_ref[...], preferred_element_type=jnp.float32) * dinv[:, None]
    )


def _tc3_body(p_ref, h2_ref, degp_ref, b2_ref, wl_ref, bl_ref, out_ref):
    dinv = _dinv_from(degp_ref)
    z = (p_ref[0] + p_ref[1] + h2_ref[...]) * dinv[:, None] + b2_ref[...]
    a = jnp.where(z >= 0, z, 0.01 * z)
    logits = jnp.dot(a, wl_ref[...], preferred_element_type=jnp.float32) + bl_ref[...]
    m = jnp.max(logits, axis=1, keepdims=True)
    lse = jnp.log(jnp.sum(jnp.exp(logits - m), axis=1, keepdims=True)) + m
    out_ref[...] = logits - lse


def _tc1(x, degp):
    return pl.pallas_call(
        _tc1_body,
        grid=(G,),
        in_specs=[
            pl.BlockSpec((R, 128), lambda i: (i, 0)),
            pl.BlockSpec((NC, R, 1), lambda i: (0, i, 0)),
        ],
        out_specs=pl.BlockSpec((R, D), lambda i: (i, 0)),
        out_shape=jax.ShapeDtypeStruct((N, D), jnp.float32),
    )(x, degp)


def _tc2(p1, y, degp, w1, b1r, w2p):
    return pl.pallas_call(
        _tc2_body,
        grid=(G,),
        in_specs=[
            pl.BlockSpec((NC, R, D), lambda i: (0, i, 0)),
            pl.BlockSpec((R, D), lambda i: (i, 0)),
            pl.BlockSpec((NC, R, 1), lambda i: (0, i, 0)),
            pl.BlockSpec((128, DH1), lambda i: (0, 0)),
            pl.BlockSpec((1, DH1), lambda i: (0, 0)),
            pl.BlockSpec((DH1, D), lambda i: (0, 0)),
        ],
        out_specs=pl.BlockSpec((R, D), lambda i: (i, 0)),
        out_shape=jax.ShapeDtypeStruct((N, D), jnp.float32),
    )(p1, y, degp, w1, b1r, w2p)


def _tc3(p2, h2, degp, b2p, wlp, blp):
    return pl.pallas_call(
        _tc3_body,
        grid=(G,),
        in_specs=[
            pl.BlockSpec((NC, R, D), lambda i: (0, i, 0)),
            pl.BlockSpec((R, D), lambda i: (i, 0)),
            pl.BlockSpec((NC, R, 1), lambda i: (0, i, 0)),
            pl.BlockSpec((1, D), lambda i: (0, 0)),
            pl.BlockSpec((D, NCLS), lambda i: (0, 0)),
            pl.BlockSpec((1, NCLS), lambda i: (0, 0)),
        ],
        out_specs=pl.BlockSpec((R, NCLS), lambda i: (i, 0)),
        out_shape=jax.ShapeDtypeStruct((N, NCLS), jnp.float32),
    )(p2, h2, degp, b2p, wlp, blp)


def kernel(x, edge_index, W1, b1, W2, b2, Wl, bl):
    src = edge_index[0].reshape(NW, EPW)
    dst = edge_index[1].reshape(NW, EPW)
    npad = EPAD - EPW
    padc = jnp.arange(npad, dtype=jnp.int32) % 16
    src_g = jnp.concatenate(
        [src, jnp.broadcast_to(padc, (NW, npad))], axis=1
    ).reshape(NW, PHASES, EPP)
    dst_g = jnp.concatenate(
        [dst, jnp.broadcast_to(N + padc, (NW, npad))], axis=1
    ).reshape(NW, PHASES, PCH, CH)

    dst_f = dst_g.reshape(NW, PHASES, EPP)
    zacc = jnp.zeros((NROWS, D), jnp.float32)

    b1r = b1.reshape(1, DH1)
    w2p = jnp.pad(W2, ((0, 0), (0, D - W2.shape[1])))
    b2p = jnp.pad(b2, (0, D - b2.shape[0])).reshape(1, D)
    wlp = jnp.pad(Wl, ((0, D - Wl.shape[0]), (0, 0)))
    blp = bl.reshape(1, NCLS)

    degh = _deg_pass(dst_f, zacc)
    degp = degh.reshape(NC, HR * 128)[:, :N].reshape(NC, N, 1)
    y = _tc1(x, degp)
    p1 = _edge_pass(y, src_g, dst_g, zacc)
    h2 = _tc2(p1, y, degp, W1, b1r, w2p)
    p2 = _edge_pass(h2, src_g, dst_g, zacc)
    return _tc3(p2, h2, degp, b2p, wlp, blp)
